# Initial kernel scaffold; baseline (speedup 1.0000x reference)
import jax
import jax.numpy as jnp
from jax.experimental import pallas as pl


def kernel(input_embeds, grid_sizes, image_embedding, W1, b1, W2, b2):
    raise NotImplementedError("write your pallas kernel here")


# speedup vs baseline: 1.1429x; 1.1429x over previous
import functools

import numpy as np
import jax
import jax.numpy as jnp
from jax import lax
from jax.experimental import pallas as pl
from jax.experimental.pallas import tpu as pltpu
from jax.experimental.pallas import tpu_sc as plsc

_N = 16
_D = 512
_FOUR_D = 4 * _D
_TOKENS = 1376
_TOKENS_PAD = 1408
_GATHERS = _TOKENS_PAD * 4
_TABLE_ROWS = 5205
_IMG_ROW = 5200
_ZERO_ROW = 5204

_NW = 32
_ROWS_PER_W = _GATHERS // _NW
_GATHER_SPLIT = 2
_ROWS_PER_DMA = _ROWS_PER_W // _GATHER_SPLIT


def _build_gather_indices() -> np.ndarray:
    idx = []
    off = 0
    for i in range(_N):
        h, w = 2 * i, 2 * i + 1
        idx.extend([_IMG_ROW, _IMG_ROW + 1, _IMG_ROW + 2, _IMG_ROW + 3])
        for r in range(h // 2):
            for c in range((w + 1) // 2):
                idx.append(off + (2 * r) * w + 2 * c)
                idx.append(off + (2 * r) * w + 2 * c + 1 if 2 * c + 1 < w else _ZERO_ROW)
                idx.append(off + (2 * r + 1) * w + 2 * c)
                idx.append(off + (2 * r + 1) * w + 2 * c + 1 if 2 * c + 1 < w else _ZERO_ROW)
        off += h * w
    while len(idx) < _GATHERS:
        idx.append(_ZERO_ROW)
    return np.asarray(idx, np.int32).reshape(_NW, _GATHER_SPLIT, _ROWS_PER_DMA)


_GATHER_IDX = _build_gather_indices()


def _sc_gather(table: jax.Array, idx: jax.Array) -> jax.Array:
    mesh = plsc.VectorSubcoreMesh(core_axis_name="c", subcore_axis_name="s")

    @functools.partial(
        pl.kernel,
        mesh=mesh,
        out_type=jax.ShapeDtypeStruct((_GATHERS, _D), jnp.float32),
        scratch_types=[
            pltpu.VMEM((_GATHER_SPLIT, _ROWS_PER_DMA), jnp.int32),
            pltpu.VMEM((_ROWS_PER_W, _D), jnp.float32),
            pltpu.SemaphoreType.DMA,
        ],
    )
    def k(table_hbm, idx_hbm, out_hbm, idx_v, rows_v, sem):
        wid = lax.axis_index("s") * 2 + lax.axis_index("c")
        pltpu.sync_copy(idx_hbm.at[wid], idx_v)
        copies = []
        for j in range(_GATHER_SPLIT):
            copies.append(pltpu.async_copy(
                table_hbm.at[idx_v.at[j]],
                rows_v.at[pl.ds(j * _ROWS_PER_DMA, _ROWS_PER_DMA)],
                sem))
        for c in copies:
            c.wait()
        pltpu.sync_copy(rows_v, out_hbm.at[pl.ds(wid * _ROWS_PER_W, _ROWS_PER_W)])

    return k(table, idx)


_BM = 128


def _mlp_body(e_ref, w1_ref, b1_ref, w2_ref, b2_ref, out_ref):
    h = lax.dot_general(e_ref[...], w1_ref[...], (((1,), (1,)), ((), ())),
                        preferred_element_type=jnp.float32)
    h = h + b1_ref[...]
    h = 0.5 * h * (1.0 + lax.erf(h * np.float32(0.7071067811865476)))
    out = lax.dot_general(h, w2_ref[...], (((1,), (1,)), ((), ())),
                          preferred_element_type=jnp.float32)
    out_ref[...] = out + b2_ref[...]


def _tc_mlp(e: jax.Array, W1, b1, W2, b2) -> jax.Array:
    grid = (_TOKENS_PAD // _BM,)
    return pl.pallas_call(
        _mlp_body,
        grid=grid,
        in_specs=[
            pl.BlockSpec((_BM, _FOUR_D), lambda i: (i, 0)),
            pl.BlockSpec((2048, _FOUR_D), lambda i: (0, 0)),
            pl.BlockSpec((1, 2048), lambda i: (0, 0)),
            pl.BlockSpec((2048, 2048), lambda i: (0, 0)),
            pl.BlockSpec((1, 2048), lambda i: (0, 0)),
        ],
        out_specs=pl.BlockSpec((_BM, 2048), lambda i: (i, 0)),
        out_shape=jax.ShapeDtypeStruct((_TOKENS_PAD, 2048), jnp.float32),
    )(e, W1, b1.reshape(1, 2048), W2, b2.reshape(1, 2048))


def kernel(input_embeds, grid_sizes, image_embedding, W1, b1, W2, b2):
    del grid_sizes
    table = jnp.concatenate(
        [input_embeds,
         image_embedding.reshape(4, _D),
         jnp.zeros((1, _D), jnp.float32)], axis=0)
    idx = jnp.asarray(_GATHER_IDX)
    e = _sc_gather(table, idx).reshape(_TOKENS_PAD, _FOUR_D)
    out = _tc_mlp(e, W1, b1, W2, b2)
    return out[:_TOKENS]


# speedup vs baseline: 1.1583x; 1.0134x over previous
import functools

import numpy as np
import jax
import jax.numpy as jnp
from jax import lax
from jax.experimental import pallas as pl
from jax.experimental.pallas import tpu as pltpu
from jax.experimental.pallas import tpu_sc as plsc

_N = 16
_D = 512
_FOUR_D = 4 * _D
_TOKENS = 1376
_TOKENS_PAD = 1408
_GATHERS = _TOKENS_PAD * 4
_TABLE_ROWS = 5205
_IMG_ROW = 5200
_ZERO_ROW = 5204

_NW = 32
_ROWS_PER_W = _GATHERS // _NW
_GATHER_SPLIT = 2
_ROWS_PER_DMA = _ROWS_PER_W // _GATHER_SPLIT


def _build_gather_indices() -> np.ndarray:
    idx = []
    off = 0
    for i in range(_N):
        h, w = 2 * i, 2 * i + 1
        idx.extend([_IMG_ROW, _IMG_ROW + 1, _IMG_ROW + 2, _IMG_ROW + 3])
        for r in range(h // 2):
            for c in range((w + 1) // 2):
                idx.append(off + (2 * r) * w + 2 * c)
                idx.append(off + (2 * r) * w + 2 * c + 1 if 2 * c + 1 < w else _ZERO_ROW)
                idx.append(off + (2 * r + 1) * w + 2 * c)
                idx.append(off + (2 * r + 1) * w + 2 * c + 1 if 2 * c + 1 < w else _ZERO_ROW)
        off += h * w
    while len(idx) < _GATHERS:
        idx.append(_ZERO_ROW)
    return np.asarray(idx, np.int32).reshape(_NW, _GATHER_SPLIT, _ROWS_PER_DMA)


_GATHER_IDX = _build_gather_indices()


def _sc_gather(table: jax.Array, idx: jax.Array) -> jax.Array:
    mesh = plsc.VectorSubcoreMesh(core_axis_name="c", subcore_axis_name="s")

    @functools.partial(
        pl.kernel,
        mesh=mesh,
        out_type=jax.ShapeDtypeStruct((_GATHERS, _D), jnp.float32),
        scratch_types=[
            pltpu.VMEM((_GATHER_SPLIT, _ROWS_PER_DMA), jnp.int32),
            pltpu.VMEM((_ROWS_PER_W, _D), jnp.float32),
            pltpu.SemaphoreType.DMA,
        ],
    )
    def k(table_hbm, idx_hbm, out_hbm, idx_v, rows_v, sem):
        wid = lax.axis_index("s") * 2 + lax.axis_index("c")
        pltpu.sync_copy(idx_hbm.at[wid], idx_v)
        copies = []
        for j in range(_GATHER_SPLIT):
            copies.append(pltpu.async_copy(
                table_hbm.at[idx_v.at[j]],
                rows_v.at[pl.ds(j * _ROWS_PER_DMA, _ROWS_PER_DMA)],
                sem))
        for c in copies:
            c.wait()
        pltpu.sync_copy(rows_v, out_hbm.at[pl.ds(wid * _ROWS_PER_W, _ROWS_PER_W)])

    return k(table, idx)


_BM = 128


def _mlp_body(e_ref, w1_ref, b1_ref, w2_ref, b2_ref, out_ref):
    h = lax.dot_general(e_ref[...].astype(jnp.bfloat16), w1_ref[...],
                        (((1,), (1,)), ((), ())),
                        preferred_element_type=jnp.float32)
    h = h + b1_ref[...]
    h = 0.5 * h * (1.0 + lax.erf(h * np.float32(0.7071067811865476)))
    out = lax.dot_general(h.astype(jnp.bfloat16), w2_ref[...],
                          (((1,), (1,)), ((), ())),
                          preferred_element_type=jnp.float32)
    out_ref[...] = out + b2_ref[...]


def _tc_mlp(e: jax.Array, W1, b1, W2, b2) -> jax.Array:
    grid = (_TOKENS_PAD // _BM,)
    return pl.pallas_call(
        _mlp_body,
        grid=grid,
        in_specs=[
            pl.BlockSpec((_BM, _FOUR_D), lambda i: (i, 0)),
            pl.BlockSpec((2048, _FOUR_D), lambda i: (0, 0)),
            pl.BlockSpec((1, 2048), lambda i: (0, 0)),
            pl.BlockSpec((2048, 2048), lambda i: (0, 0)),
            pl.BlockSpec((1, 2048), lambda i: (0, 0)),
        ],
        out_specs=pl.BlockSpec((_BM, 2048), lambda i: (i, 0)),
        out_shape=jax.ShapeDtypeStruct((_TOKENS, 2048), jnp.float32),
    )(e, W1.astype(jnp.bfloat16), b1.reshape(1, 2048),
      W2.astype(jnp.bfloat16), b2.reshape(1, 2048))


def kernel(input_embeds, grid_sizes, image_embedding, W1, b1, W2, b2):
    del grid_sizes
    table = jnp.concatenate(
        [input_embeds,
         image_embedding.reshape(4, _D),
         jnp.zeros((1, _D), jnp.float32)], axis=0)
    idx = jnp.asarray(_GATHER_IDX)
    e = _sc_gather(table, idx).reshape(_TOKENS_PAD, _FOUR_D)
    return _tc_mlp(e, W1, b1, W2, b2)


# speedup vs baseline: 1.2495x; 1.0788x over previous
import functools

import numpy as np
import jax
import jax.numpy as jnp
from jax import lax
from jax.experimental import pallas as pl
from jax.experimental.pallas import tpu as pltpu
from jax.experimental.pallas import tpu_sc as plsc

_N = 16
_D = 512
_TOKENS = 1376
_TOKENS_PAD = 1408
_GATHERS = _TOKENS_PAD * 4
_IMG_ROW = 5200
_ZERO_ROW = 5204

_NW = 32
_ROWS_PER_W = _GATHERS // _NW
_GATHER_SPLIT = 2
_ROWS_PER_DMA = _ROWS_PER_W // _GATHER_SPLIT


def _build_gather_indices() -> np.ndarray:
    idx = []
    off = 0
    for i in range(_N):
        h, w = 2 * i, 2 * i + 1
        idx.extend([_IMG_ROW, _IMG_ROW + 1, _IMG_ROW + 2, _IMG_ROW + 3])
        for r in range(h // 2):
            for c in range((w + 1) // 2):
                idx.append(off + (2 * r) * w + 2 * c)
                idx.append(off + (2 * r) * w + 2 * c + 1 if 2 * c + 1 < w else _ZERO_ROW)
                idx.append(off + (2 * r + 1) * w + 2 * c)
                idx.append(off + (2 * r + 1) * w + 2 * c + 1 if 2 * c + 1 < w else _ZERO_ROW)
        off += h * w
    while len(idx) < _GATHERS:
        idx.append(_ZERO_ROW)
    token_major = np.asarray(idx, np.int32).reshape(_TOKENS_PAD, 4)
    group_major = token_major.T.copy()
    return group_major.reshape(_NW, _GATHER_SPLIT, _ROWS_PER_DMA)


_GATHER_IDX = _build_gather_indices()


def _sc_gather(table: jax.Array, idx: jax.Array) -> jax.Array:
    mesh = plsc.VectorSubcoreMesh(core_axis_name="c", subcore_axis_name="s")

    @functools.partial(
        pl.kernel,
        mesh=mesh,
        out_type=jax.ShapeDtypeStruct((_GATHERS, _D), jnp.float32),
        scratch_types=[
            pltpu.VMEM((_GATHER_SPLIT, _ROWS_PER_DMA), jnp.int32),
            pltpu.VMEM((_ROWS_PER_W, _D), jnp.float32),
            pltpu.SemaphoreType.DMA,
        ],
    )
    def k(table_hbm, idx_hbm, out_hbm, idx_v, rows_v, sem):
        wid = lax.axis_index("s") * 2 + lax.axis_index("c")
        pltpu.sync_copy(idx_hbm.at[wid], idx_v)
        copies = []
        for j in range(_GATHER_SPLIT):
            copies.append(pltpu.async_copy(
                table_hbm.at[idx_v.at[j]],
                rows_v.at[pl.ds(j * _ROWS_PER_DMA, _ROWS_PER_DMA)],
                sem))
        for c in copies:
            c.wait()
        pltpu.sync_copy(rows_v, out_hbm.at[pl.ds(wid * _ROWS_PER_W, _ROWS_PER_W)])

    return k(table, idx)


_BM = 128
_NBLK = _TOKENS_PAD // _BM


def _mlp_body(e0_ref, e1_ref, e2_ref, e3_ref, w1_ref, b1_ref, w2_ref, b2_ref,
              out_ref):
    h = lax.dot_general(e0_ref[...].astype(jnp.bfloat16), w1_ref[:, 0:_D],
                        (((1,), (1,)), ((), ())),
                        preferred_element_type=jnp.float32)
    for k, e_ref in ((1, e1_ref), (2, e2_ref), (3, e3_ref)):
        h += lax.dot_general(e_ref[...].astype(jnp.bfloat16),
                             w1_ref[:, k * _D:(k + 1) * _D],
                             (((1,), (1,)), ((), ())),
                             preferred_element_type=jnp.float32)
    h = h + b1_ref[...]
    h = 0.5 * h * (1.0 + lax.erf(h * np.float32(0.7071067811865476)))
    out = lax.dot_general(h.astype(jnp.bfloat16), w2_ref[...],
                          (((1,), (1,)), ((), ())),
                          preferred_element_type=jnp.float32)
    out_ref[...] = out + b2_ref[...]


def _tc_mlp(e4: jax.Array, W1, b1, W2, b2) -> jax.Array:
    espec = lambda k: pl.BlockSpec((_BM, _D), lambda i, _k=k: (_k * _NBLK + i, 0))
    return pl.pallas_call(
        _mlp_body,
        grid=(_NBLK,),
        in_specs=[
            espec(0), espec(1), espec(2), espec(3),
            pl.BlockSpec((2048, 2048), lambda i: (0, 0)),
            pl.BlockSpec((1, 2048), lambda i: (0, 0)),
            pl.BlockSpec((2048, 2048), lambda i: (0, 0)),
            pl.BlockSpec((1, 2048), lambda i: (0, 0)),
        ],
        out_specs=pl.BlockSpec((_BM, 2048), lambda i: (i, 0)),
        out_shape=jax.ShapeDtypeStruct((_TOKENS, 2048), jnp.float32),
    )(e4, e4, e4, e4, W1.astype(jnp.bfloat16), b1.reshape(1, 2048),
      W2.astype(jnp.bfloat16), b2.reshape(1, 2048))


def kernel(input_embeds, grid_sizes, image_embedding, W1, b1, W2, b2):
    del grid_sizes
    table = jnp.concatenate(
        [input_embeds,
         image_embedding.reshape(4, _D),
         jnp.zeros((1, _D), jnp.float32)], axis=0)
    idx = jnp.asarray(_GATHER_IDX)
    e4 = _sc_gather(table, idx)
    return _tc_mlp(e4, W1, b1, W2, b2)


# speedup vs baseline: 1.4045x; 1.1240x over previous
import functools

import numpy as np
import jax
import jax.numpy as jnp
from jax import lax
from jax.experimental import pallas as pl
from jax.experimental.pallas import tpu as pltpu
from jax.experimental.pallas import tpu_sc as plsc

_N = 16
_D = 512
_TOKENS = 1376
_TOKENS_PAD = 1536
_GATHERS = _TOKENS_PAD * 4

_NW = 32
_ROWS_PER_W = _GATHERS // _NW
_GATHER_SPLIT = 2
_ROWS_PER_DMA = _ROWS_PER_W // _GATHER_SPLIT


def _build_static_layout():
    src = np.zeros((_TOKENS_PAD, 4), np.int32)
    keep = np.zeros((_TOKENS_PAD, 4), np.float32)
    img = np.zeros((_TOKENS_PAD,), np.float32)
    off = 0
    t = 0
    for i in range(_N):
        h, w = 2 * i, 2 * i + 1
        img[t] = 1.0
        t += 1
        for r in range(h // 2):
            for c in range((w + 1) // 2):
                for k, (dr, dc) in enumerate(((0, 0), (0, 1), (1, 0), (1, 1))):
                    rr, cc = 2 * r + dr, 2 * c + dc
                    if cc < w:
                        src[t, k] = off + rr * w + cc
                        keep[t, k] = 1.0
                t += 1
        off += h * w
    assert t == _TOKENS
    gm_src = src.T.copy().reshape(_NW, _GATHER_SPLIT, _ROWS_PER_DMA)
    gm_keep = keep.T.copy().reshape(_GATHERS, 1)
    return gm_src, gm_keep, img.reshape(_TOKENS_PAD, 1)


_GATHER_IDX, _ZMASK, _IMG_IND = _build_static_layout()


def _sc_gather(embeds: jax.Array, idx: jax.Array) -> jax.Array:
    mesh = plsc.VectorSubcoreMesh(core_axis_name="c", subcore_axis_name="s")

    @functools.partial(
        pl.kernel,
        mesh=mesh,
        out_type=jax.ShapeDtypeStruct((_GATHERS, _D), jnp.float32),
        scratch_types=[
            pltpu.VMEM((_GATHER_SPLIT, _ROWS_PER_DMA), jnp.int32),
            pltpu.VMEM((_ROWS_PER_W, _D), jnp.float32),
            pltpu.SemaphoreType.DMA,
        ],
    )
    def k(embeds_hbm, idx_hbm, out_hbm, idx_v, rows_v, sem):
        wid = lax.axis_index("s") * 2 + lax.axis_index("c")
        pltpu.sync_copy(idx_hbm.at[wid], idx_v)
        copies = []
        for j in range(_GATHER_SPLIT):
            copies.append(pltpu.async_copy(
                embeds_hbm.at[idx_v.at[j]],
                rows_v.at[pl.ds(j * _ROWS_PER_DMA, _ROWS_PER_DMA)],
                sem))
        for c in copies:
            c.wait()
        pltpu.sync_copy(rows_v, out_hbm.at[pl.ds(wid * _ROWS_PER_W, _ROWS_PER_W)])

    return k(embeds, idx)


_BM = 256
_NBLK = _TOKENS_PAD // _BM


def _mlp_body(e0_ref, e1_ref, e2_ref, e3_ref, z0_ref, z1_ref, z2_ref, z3_ref,
              img_ref, imge_ref, w1_ref, b1_ref, w2_ref, b2_ref, out_ref):
    z_refs = (z0_ref, z1_ref, z2_ref, z3_ref)
    h = None
    for k, e_ref in enumerate((e0_ref, e1_ref, e2_ref, e3_ref)):
        g = e_ref[...].astype(jnp.bfloat16) * z_refs[k][...]
        d = lax.dot_general(g, w1_ref[:, k * _D:(k + 1) * _D],
                            (((1,), (1,)), ((), ())),
                            preferred_element_type=jnp.float32)
        h = d if h is None else h + d
    h_img = lax.dot_general(imge_ref[...].astype(jnp.bfloat16), w1_ref[...],
                            (((1,), (1,)), ((), ())),
                            preferred_element_type=jnp.float32)
    h = h + img_ref[...] * h_img
    h = h + b1_ref[...]
    h = 0.5 * h * (1.0 + lax.erf(h * np.float32(0.7071067811865476)))
    out = lax.dot_general(h.astype(jnp.bfloat16), w2_ref[...],
                          (((1,), (1,)), ((), ())),
                          preferred_element_type=jnp.float32)
    out_ref[...] = out + b2_ref[...]


def _tc_mlp(e4: jax.Array, image_embedding, W1, b1, W2, b2) -> jax.Array:
    espec = lambda k: pl.BlockSpec((_BM, _D), lambda i, _k=k: (_k * _NBLK + i, 0))
    zspec = lambda k: pl.BlockSpec((_BM, 1), lambda i, _k=k: (_k * _NBLK + i, 0))
    full = lambda r, c: pl.BlockSpec((r, c), lambda i: (0, 0))
    return pl.pallas_call(
        _mlp_body,
        grid=(_NBLK,),
        in_specs=[
            espec(0), espec(1), espec(2), espec(3),
            zspec(0), zspec(1), zspec(2), zspec(3),
            pl.BlockSpec((_BM, 1), lambda i: (i, 0)),
            full(1, 2048),
            full(2048, 2048),
            full(1, 2048),
            full(2048, 2048),
            full(1, 2048),
        ],
        out_specs=pl.BlockSpec((_BM, 2048), lambda i: (i, 0)),
        out_shape=jax.ShapeDtypeStruct((_TOKENS, 2048), jnp.float32),
    )(e4, e4, e4, e4,
      jnp.asarray(_ZMASK, jnp.bfloat16), jnp.asarray(_ZMASK, jnp.bfloat16),
      jnp.asarray(_ZMASK, jnp.bfloat16), jnp.asarray(_ZMASK, jnp.bfloat16),
      jnp.asarray(_IMG_IND, jnp.float32),
      image_embedding,
      W1.astype(jnp.bfloat16), b1.reshape(1, 2048),
      W2.astype(jnp.bfloat16), b2.reshape(1, 2048))


def kernel(input_embeds, grid_sizes, image_embedding, W1, b1, W2, b2):
    del grid_sizes
    idx = jnp.asarray(_GATHER_IDX)
    e4 = _sc_gather(input_embeds, idx)
    return _tc_mlp(e4, image_embedding, W1, b1, W2, b2)


# speedup vs baseline: 2.2103x; 1.5737x over previous
import functools

import numpy as np
import jax
import jax.numpy as jnp
from jax import lax
from jax.experimental import pallas as pl
from jax.experimental.pallas import tpu as pltpu
from jax.experimental.pallas import tpu_sc as plsc

_N = 16
_D = 512
_TOKENS = 1376
_TOKENS_PAD = 1536
_GATHERS = _TOKENS_PAD * 4

_NW = 32
_ROWS_PER_W = _GATHERS // _NW
_GATHER_SPLIT = 2
_ROWS_PER_DMA = _ROWS_PER_W // _GATHER_SPLIT


def _build_static_layout():
    spread = (np.arange(_TOKENS_PAD * 4, dtype=np.int64).reshape(4, _TOKENS_PAD).T
              % 5200).astype(np.int32)
    src = spread.copy()
    keep = np.zeros((_TOKENS_PAD, 4), np.float32)
    img = np.zeros((_TOKENS_PAD,), np.float32)
    off = 0
    t = 0
    for i in range(_N):
        h, w = 2 * i, 2 * i + 1
        img[t] = 1.0
        t += 1
        for r in range(h // 2):
            for c in range((w + 1) // 2):
                for k, (dr, dc) in enumerate(((0, 0), (0, 1), (1, 0), (1, 1))):
                    rr, cc = 2 * r + dr, 2 * c + dc
                    if cc < w:
                        src[t, k] = off + rr * w + cc
                        keep[t, k] = 1.0
                t += 1
        off += h * w
    assert t == _TOKENS
    gm_src = src.T.copy().reshape(_NW, _GATHER_SPLIT, _ROWS_PER_DMA)
    gm_keep = keep.T.copy().reshape(_GATHERS, 1)
    return gm_src, gm_keep, img.reshape(_TOKENS_PAD, 1)


_GATHER_IDX, _ZMASK, _IMG_IND = _build_static_layout()


def _sc_gather(embeds: jax.Array, idx: jax.Array) -> jax.Array:
    mesh = plsc.VectorSubcoreMesh(core_axis_name="c", subcore_axis_name="s")

    @functools.partial(
        pl.kernel,
        mesh=mesh,
        out_type=jax.ShapeDtypeStruct((_GATHERS, _D), jnp.float32),
        compiler_params=pltpu.CompilerParams(use_tc_tiling_on_sc=True),
        scratch_types=[
            pltpu.VMEM((_GATHER_SPLIT, _ROWS_PER_DMA), jnp.int32),
            pltpu.VMEM((_ROWS_PER_W, _D), jnp.float32),
            pltpu.SemaphoreType.DMA,
        ],
    )
    def k(embeds_hbm, idx_hbm, out_hbm, idx_v, rows_v, sem):
        wid = lax.axis_index("s") * 2 + lax.axis_index("c")
        pltpu.sync_copy(idx_hbm.at[wid], idx_v)
        copies = []
        for j in range(_GATHER_SPLIT):
            copies.append(pltpu.async_copy(
                embeds_hbm.at[idx_v.at[j]],
                rows_v.at[pl.ds(j * _ROWS_PER_DMA, _ROWS_PER_DMA)],
                sem))
        for c in copies:
            c.wait()
        pltpu.sync_copy(rows_v, out_hbm.at[pl.ds(wid * _ROWS_PER_W, _ROWS_PER_W)])

    return k(embeds, idx)


_BM = 256
_NBLK = _TOKENS_PAD // _BM


def _mlp_body(e0_ref, e1_ref, e2_ref, e3_ref, z0_ref, z1_ref, z2_ref, z3_ref,
              img_ref, imge_ref, w1_ref, b1_ref, w2_ref, b2_ref, out_ref):
    z_refs = (z0_ref, z1_ref, z2_ref, z3_ref)
    h = None
    for k, e_ref in enumerate((e0_ref, e1_ref, e2_ref, e3_ref)):
        g = e_ref[...].astype(jnp.bfloat16) * z_refs[k][...]
        d = lax.dot_general(g, w1_ref[:, k * _D:(k + 1) * _D],
                            (((1,), (1,)), ((), ())),
                            preferred_element_type=jnp.float32)
        h = d if h is None else h + d
    h_img = lax.dot_general(imge_ref[...].astype(jnp.bfloat16), w1_ref[...],
                            (((1,), (1,)), ((), ())),
                            preferred_element_type=jnp.float32)
    h = h + img_ref[...] * h_img
    h = h + b1_ref[...]
    h = 0.5 * h * (1.0 + lax.erf(h * np.float32(0.7071067811865476)))
    out = lax.dot_general(h.astype(jnp.bfloat16), w2_ref[...],
                          (((1,), (1,)), ((), ())),
                          preferred_element_type=jnp.float32)
    out_ref[...] = out + b2_ref[...]


def _tc_mlp(e4: jax.Array, image_embedding, W1, b1, W2, b2) -> jax.Array:
    espec = lambda k: pl.BlockSpec((_BM, _D), lambda i, _k=k: (_k * _NBLK + i, 0))
    zspec = lambda k: pl.BlockSpec((_BM, 1), lambda i, _k=k: (_k * _NBLK + i, 0))
    full = lambda r, c: pl.BlockSpec((r, c), lambda i: (0, 0))
    return pl.pallas_call(
        _mlp_body,
        grid=(_NBLK,),
        in_specs=[
            espec(0), espec(1), espec(2), espec(3),
            zspec(0), zspec(1), zspec(2), zspec(3),
            pl.BlockSpec((_BM, 1), lambda i: (i, 0)),
            full(1, 2048),
            full(2048, 2048),
            full(1, 2048),
            full(2048, 2048),
            full(1, 2048),
        ],
        out_specs=pl.BlockSpec((_BM, 2048), lambda i: (i, 0)),
        out_shape=jax.ShapeDtypeStruct((_TOKENS, 2048), jnp.float32),
    )(e4, e4, e4, e4,
      jnp.asarray(_ZMASK, jnp.bfloat16), jnp.asarray(_ZMASK, jnp.bfloat16),
      jnp.asarray(_ZMASK, jnp.bfloat16), jnp.asarray(_ZMASK, jnp.bfloat16),
      jnp.asarray(_IMG_IND, jnp.float32),
      image_embedding,
      W1.astype(jnp.bfloat16), b1.reshape(1, 2048),
      W2.astype(jnp.bfloat16), b2.reshape(1, 2048))


def kernel(input_embeds, grid_sizes, image_embedding, W1, b1, W2, b2):
    del grid_sizes
    idx = jnp.asarray(_GATHER_IDX)
    e4 = _sc_gather(input_embeds, idx)
    return _tc_mlp(e4, image_embedding, W1, b1, W2, b2)


# speedup vs baseline: 2.4766x; 1.1205x over previous
import functools

import numpy as np
import jax
import jax.numpy as jnp
from jax import lax
from jax.experimental import pallas as pl
from jax.experimental.pallas import tpu as pltpu
from jax.experimental.pallas import tpu_sc as plsc

_N = 16
_D = 512
_TOKENS = 1376
_TOKENS_PAD = 1536
_GATHERS = _TOKENS_PAD * 4

_NW = 32
_ROWS_PER_W = _GATHERS // _NW
_GATHER_SPLIT = 2
_ROWS_PER_DMA = _ROWS_PER_W // _GATHER_SPLIT


def _build_static_layout():
    spread = (np.arange(_TOKENS_PAD * 4, dtype=np.int64).reshape(4, _TOKENS_PAD).T
              % 5200).astype(np.int32)
    src = spread.copy()
    keep = np.zeros((_TOKENS_PAD, 4), np.float32)
    img = np.zeros((_TOKENS_PAD,), np.float32)
    off = 0
    t = 0
    for i in range(_N):
        h, w = 2 * i, 2 * i + 1
        img[t] = 1.0
        t += 1
        for r in range(h // 2):
            for c in range((w + 1) // 2):
                for k, (dr, dc) in enumerate(((0, 0), (0, 1), (1, 0), (1, 1))):
                    rr, cc = 2 * r + dr, 2 * c + dc
                    if cc < w:
                        src[t, k] = off + rr * w + cc
                        keep[t, k] = 1.0
                t += 1
        off += h * w
    assert t == _TOKENS
    gm_src = src.T.copy().reshape(_NW, _GATHER_SPLIT, _ROWS_PER_DMA)
    gm_keep = keep.T.copy().reshape(_GATHERS, 1)
    return gm_src, gm_keep, img.reshape(_TOKENS_PAD, 1)


_GATHER_IDX, _ZMASK, _IMG_IND = _build_static_layout()


def _sc_gather(embeds: jax.Array, idx: jax.Array) -> jax.Array:
    mesh = plsc.VectorSubcoreMesh(core_axis_name="c", subcore_axis_name="s")

    @functools.partial(
        pl.kernel,
        mesh=mesh,
        out_type=jax.ShapeDtypeStruct((_GATHERS, _D), jnp.float32),
        compiler_params=pltpu.CompilerParams(use_tc_tiling_on_sc=True),
        scratch_types=[
            pltpu.VMEM((_GATHER_SPLIT, _ROWS_PER_DMA), jnp.int32),
            pltpu.VMEM((_ROWS_PER_W, _D), jnp.float32),
            pltpu.SemaphoreType.DMA,
        ],
    )
    def k(embeds_hbm, idx_hbm, out_hbm, idx_v, rows_v, sem):
        wid = lax.axis_index("s") * 2 + lax.axis_index("c")
        pltpu.sync_copy(idx_hbm.at[wid], idx_v)
        copies = []
        for j in range(_GATHER_SPLIT):
            copies.append(pltpu.async_copy(
                embeds_hbm.at[idx_v.at[j]],
                rows_v.at[pl.ds(j * _ROWS_PER_DMA, _ROWS_PER_DMA)],
                sem))
        for c in copies:
            c.wait()
        pltpu.sync_copy(rows_v, out_hbm.at[pl.ds(wid * _ROWS_PER_W, _ROWS_PER_W)])

    return k(embeds, idx)


_BM = 512
_NBLK = _TOKENS_PAD // _BM


def _mlp_body(e0_ref, e1_ref, e2_ref, e3_ref, z0_ref, z1_ref, z2_ref, z3_ref,
              img_ref, imge_ref, w1_ref, b1_ref, w2_ref, b2_ref, out_ref,
              himg_ref):
    @pl.when(pl.program_id(0) == 0)
    def _():
        himg_ref[...] = lax.dot_general(
            imge_ref[...].astype(jnp.bfloat16), w1_ref[...],
            (((1,), (1,)), ((), ())), preferred_element_type=jnp.float32)

    z_refs = (z0_ref, z1_ref, z2_ref, z3_ref)
    h = None
    for k, e_ref in enumerate((e0_ref, e1_ref, e2_ref, e3_ref)):
        g = e_ref[...].astype(jnp.bfloat16) * z_refs[k][...]
        d = lax.dot_general(g, w1_ref[:, k * _D:(k + 1) * _D],
                            (((1,), (1,)), ((), ())),
                            preferred_element_type=jnp.float32)
        h = d if h is None else h + d
    h = h + img_ref[...] * himg_ref[...]
    h = h + b1_ref[...]
    h = 0.5 * h * (1.0 + lax.erf(h * np.float32(0.7071067811865476)))
    out = lax.dot_general(h.astype(jnp.bfloat16), w2_ref[...],
                          (((1,), (1,)), ((), ())),
                          preferred_element_type=jnp.float32)
    out_ref[...] = out + b2_ref[...]


def _tc_mlp(e4: jax.Array, image_embedding, W1, b1, W2, b2) -> jax.Array:
    espec = lambda k: pl.BlockSpec((_BM, _D), lambda i, _k=k: (_k * _NBLK + i, 0))
    zspec = lambda k: pl.BlockSpec((_BM, 1), lambda i, _k=k: (_k * _NBLK + i, 0))
    full = lambda r, c: pl.BlockSpec((r, c), lambda i: (0, 0))
    return pl.pallas_call(
        _mlp_body,
        grid=(_NBLK,),
        in_specs=[
            espec(0), espec(1), espec(2), espec(3),
            zspec(0), zspec(1), zspec(2), zspec(3),
            pl.BlockSpec((_BM, 1), lambda i: (i, 0)),
            full(1, 2048),
            full(2048, 2048),
            full(1, 2048),
            full(2048, 2048),
            full(1, 2048),
        ],
        out_specs=pl.BlockSpec((_BM, 2048), lambda i: (i, 0)),
        out_shape=jax.ShapeDtypeStruct((_TOKENS, 2048), jnp.float32),
        scratch_shapes=[pltpu.VMEM((1, 2048), jnp.float32)],
    )(e4, e4, e4, e4,
      jnp.asarray(_ZMASK, jnp.bfloat16), jnp.asarray(_ZMASK, jnp.bfloat16),
      jnp.asarray(_ZMASK, jnp.bfloat16), jnp.asarray(_ZMASK, jnp.bfloat16),
      jnp.asarray(_IMG_IND, jnp.float32),
      image_embedding,
      W1.astype(jnp.bfloat16), b1.reshape(1, 2048),
      W2.astype(jnp.bfloat16), b2.reshape(1, 2048))


def kernel(input_embeds, grid_sizes, image_embedding, W1, b1, W2, b2):
    del grid_sizes
    idx = jnp.asarray(_GATHER_IDX)
    e4 = _sc_gather(input_embeds, idx)
    return _tc_mlp(e4, image_embedding, W1, b1, W2, b2)


# speedup vs baseline: 2.5132x; 1.0148x over previous
import functools

import numpy as np
import jax
import jax.numpy as jnp
from jax import lax
from jax.experimental import pallas as pl
from jax.experimental.pallas import tpu as pltpu
from jax.experimental.pallas import tpu_sc as plsc

_N = 16
_D = 512
_TOKENS = 1376
_TOKENS_PAD = 1536
_GATHERS = _TOKENS_PAD * 4

_NW = 32
_ROWS_PER_W = _GATHERS // _NW
_GATHER_SPLIT = 2
_ROWS_PER_DMA = _ROWS_PER_W // _GATHER_SPLIT


def _build_static_layout():
    spread = (np.arange(_TOKENS_PAD * 4, dtype=np.int64).reshape(4, _TOKENS_PAD).T
              % 5200).astype(np.int32)
    src = spread.copy()
    keep = np.zeros((_TOKENS_PAD, 4), np.float32)
    img = np.zeros((_TOKENS_PAD,), np.float32)
    off = 0
    t = 0
    for i in range(_N):
        h, w = 2 * i, 2 * i + 1
        img[t] = 1.0
        t += 1
        for r in range(h // 2):
            for c in range((w + 1) // 2):
                for k, (dr, dc) in enumerate(((0, 0), (0, 1), (1, 0), (1, 1))):
                    rr, cc = 2 * r + dr, 2 * c + dc
                    if cc < w:
                        src[t, k] = off + rr * w + cc
                        keep[t, k] = 1.0
                t += 1
        off += h * w
    assert t == _TOKENS
    gm_src = src.T.copy().reshape(_NW, _GATHER_SPLIT, _ROWS_PER_DMA)
    gm_keep = keep.T.copy().reshape(_GATHERS, 1)
    return gm_src, gm_keep, img.reshape(_TOKENS_PAD, 1)


_GATHER_IDX, _ZMASK, _IMG_IND = _build_static_layout()


def _sc_gather(embeds: jax.Array, idx: jax.Array) -> jax.Array:
    mesh = plsc.VectorSubcoreMesh(core_axis_name="c", subcore_axis_name="s")

    @functools.partial(
        pl.kernel,
        mesh=mesh,
        out_type=jax.ShapeDtypeStruct((_GATHERS, _D), jnp.float32),
        compiler_params=pltpu.CompilerParams(use_tc_tiling_on_sc=True),
        scratch_types=[
            pltpu.VMEM((_GATHER_SPLIT, _ROWS_PER_DMA), jnp.int32),
            pltpu.VMEM((_ROWS_PER_W, _D), jnp.float32),
            pltpu.SemaphoreType.DMA,
        ],
    )
    def k(embeds_hbm, idx_hbm, out_hbm, idx_v, rows_v, sem):
        wid = lax.axis_index("s") * 2 + lax.axis_index("c")
        pltpu.sync_copy(idx_hbm.at[wid], idx_v)
        copies = []
        for j in range(_GATHER_SPLIT):
            copies.append(pltpu.async_copy(
                embeds_hbm.at[idx_v.at[j]],
                rows_v.at[pl.ds(j * _ROWS_PER_DMA, _ROWS_PER_DMA)],
                sem))
        for c in copies:
            c.wait()
        pltpu.sync_copy(rows_v, out_hbm.at[pl.ds(wid * _ROWS_PER_W, _ROWS_PER_W)])

    return k(embeds, idx)


def _cast_body(w1_ref, w2_ref, o1_ref, o2_ref):
    o1_ref[...] = w1_ref[...].astype(jnp.bfloat16)
    o2_ref[...] = w2_ref[...].astype(jnp.bfloat16)


def _cast_weights(W1, W2):
    blk = lambda: pl.BlockSpec((512, 2048), lambda i: (i, 0))
    return pl.pallas_call(
        _cast_body,
        grid=(4,),
        in_specs=[blk(), blk()],
        out_specs=[blk(), blk()],
        out_shape=(jax.ShapeDtypeStruct((2048, 2048), jnp.bfloat16),
                   jax.ShapeDtypeStruct((2048, 2048), jnp.bfloat16)),
    )(W1, W2)


_BM = 512
_NBLK = _TOKENS_PAD // _BM


def _mlp_body(e0_ref, e1_ref, e2_ref, e3_ref, z0_ref, z1_ref, z2_ref, z3_ref,
              img_ref, imge_ref, w1_ref, b1_ref, w2_ref, b2_ref, out_ref,
              himg_ref):
    @pl.when(pl.program_id(0) == 0)
    def _():
        himg_ref[...] = lax.dot_general(
            imge_ref[...].astype(jnp.bfloat16), w1_ref[...],
            (((1,), (1,)), ((), ())), preferred_element_type=jnp.float32)

    z_refs = (z0_ref, z1_ref, z2_ref, z3_ref)
    h = None
    for k, e_ref in enumerate((e0_ref, e1_ref, e2_ref, e3_ref)):
        g = e_ref[...].astype(jnp.bfloat16) * z_refs[k][...]
        d = lax.dot_general(g, w1_ref[:, k * _D:(k + 1) * _D],
                            (((1,), (1,)), ((), ())),
                            preferred_element_type=jnp.float32)
        h = d if h is None else h + d
    h = h + img_ref[...] * himg_ref[...]
    h = h + b1_ref[...]
    h = 0.5 * h * (1.0 + lax.erf(h * np.float32(0.7071067811865476)))
    out = lax.dot_general(h.astype(jnp.bfloat16), w2_ref[...],
                          (((1,), (1,)), ((), ())),
                          preferred_element_type=jnp.float32)
    out_ref[...] = out + b2_ref[...]


def _tc_mlp(e4: jax.Array, image_embedding, W1bf, b1, W2bf, b2) -> jax.Array:
    espec = lambda k: pl.BlockSpec((_BM, _D), lambda i, _k=k: (_k * _NBLK + i, 0))
    zspec = lambda k: pl.BlockSpec((_BM, 1), lambda i, _k=k: (_k * _NBLK + i, 0))
    full = lambda r, c: pl.BlockSpec((r, c), lambda i: (0, 0))
    return pl.pallas_call(
        _mlp_body,
        grid=(_NBLK,),
        in_specs=[
            espec(0), espec(1), espec(2), espec(3),
            zspec(0), zspec(1), zspec(2), zspec(3),
            pl.BlockSpec((_BM, 1), lambda i: (i, 0)),
            full(1, 2048),
            full(2048, 2048),
            full(1, 2048),
            full(2048, 2048),
            full(1, 2048),
        ],
        out_specs=pl.BlockSpec((_BM, 2048), lambda i: (i, 0)),
        out_shape=jax.ShapeDtypeStruct((_TOKENS, 2048), jnp.float32),
        scratch_shapes=[pltpu.VMEM((1, 2048), jnp.float32)],
    )(e4, e4, e4, e4,
      jnp.asarray(_ZMASK, jnp.bfloat16), jnp.asarray(_ZMASK, jnp.bfloat16),
      jnp.asarray(_ZMASK, jnp.bfloat16), jnp.asarray(_ZMASK, jnp.bfloat16),
      jnp.asarray(_IMG_IND, jnp.float32),
      image_embedding,
      W1bf, b1.reshape(1, 2048),
      W2bf, b2.reshape(1, 2048))


def kernel(input_embeds, grid_sizes, image_embedding, W1, b1, W2, b2):
    del grid_sizes
    idx = jnp.asarray(_GATHER_IDX)
    e4 = _sc_gather(input_embeds, idx)
    W1bf, W2bf = _cast_weights(W1, W2)
    return _tc_mlp(e4, image_embedding, W1bf, b1, W2bf, b2)
